# baseline (speedup 1.0000x reference)
import functools

import jax
import jax.numpy as jnp
from jax import lax
from jax.experimental import pallas as pl
from jax.experimental.pallas import tpu as pltpu
from jax.experimental.pallas import tpu_sc as plsc

MAX_IDX = 10
N = 16384
LANES = 16
NUM_SUBCORES = 16
CHUNK = N // NUM_SUBCORES
NUM_VECS = CHUNK // LANES

_GATHER_DNUMS = lax.GatherDimensionNumbers(
    offset_dims=(), collapsed_slice_dims=(0,), start_index_map=(0,))


def _vgather(vec, idx):
    return lax.gather(vec, idx, _GATHER_DNUMS, slice_sizes=(1,),
                      mode=lax.GatherScatterMode.PROMISE_IN_BOUNDS)


def _make_sc_kernel():
    mesh = plsc.VectorSubcoreMesh(core_axis_name="c", subcore_axis_name="s",
                                  num_cores=1)

    @functools.partial(
        pl.kernel,
        mesh=mesh,
        out_type=jax.ShapeDtypeStruct((N,), jnp.float32),
        scratch_types=[
            pltpu.VMEM((CHUNK,), jnp.float32),
            pltpu.VMEM((CHUNK,), jnp.int32),
            pltpu.VMEM((LANES,), jnp.float32),
            pltpu.SemaphoreType.DMA,
        ],
    )
    def body(logits_hbm, counts_hbm, params_hbm, out_hbm, lv, cv, pv, sem):
        base = lax.axis_index("s") * CHUNK
        c1 = pltpu.async_copy(logits_hbm.at[pl.ds(base, CHUNK)], lv, sem)
        c2 = pltpu.async_copy(counts_hbm.at[pl.ds(base, CHUNK)], cv, sem)
        c3 = pltpu.async_copy(params_hbm, pv.at[pl.ds(0, MAX_IDX + 2)], sem)
        c1.wait()
        c2.wait()
        c3.wait()
        params = pv[...]
        lane = lax.iota(jnp.int32, LANES)
        m = _vgather(params, (lane * 0 + (MAX_IDX + 1))[:, None])
        two_inv_m = 2.0 / m
        two_m = m + m

        def step(i, carry):
            sl = pl.ds(i * LANES, LANES)
            idx = jnp.minimum(cv[sl], MAX_IDX)
            scale = _vgather(params, idx[:, None])
            e = jnp.exp(lv[sl] * scale * two_inv_m)
            lv[sl] = m - two_m / (e + 1.0)
            return carry

        lax.fori_loop(0, NUM_VECS, step, 0)
        pltpu.sync_copy(lv, out_hbm.at[pl.ds(base, CHUNK)])

    return body


_calibrate = _make_sc_kernel()


def kernel(logits, alt_counts, confidence, max_logit):
    counts = alt_counts.astype(jnp.int32)
    params = jnp.concatenate([
        confidence.astype(jnp.float32),
        jnp.reshape(max_logit, (1,)).astype(jnp.float32),
    ])
    return _calibrate(logits.astype(jnp.float32), counts, params)


# speedup vs baseline: 1.0014x; 1.0014x over previous
import functools

import jax
import jax.numpy as jnp
from jax import lax
from jax.experimental import pallas as pl
from jax.experimental.pallas import tpu as pltpu
from jax.experimental.pallas import tpu_sc as plsc

MAX_IDX = 10
N = 16384
LANES = 16
NUM_SUBCORES = 16
CHUNK = N // NUM_SUBCORES
NUM_VECS = CHUNK // LANES

_GATHER_DNUMS = lax.GatherDimensionNumbers(
    offset_dims=(), collapsed_slice_dims=(0,), start_index_map=(0,))


def _vgather(vec, idx):
    return lax.gather(vec, idx, _GATHER_DNUMS, slice_sizes=(1,),
                      mode=lax.GatherScatterMode.PROMISE_IN_BOUNDS)


def _make_sc_kernel():
    mesh = plsc.VectorSubcoreMesh(core_axis_name="c", subcore_axis_name="s",
                                  num_cores=1)

    @functools.partial(
        pl.kernel,
        mesh=mesh,
        out_type=jax.ShapeDtypeStruct((N,), jnp.float32),
        scratch_types=[
            pltpu.VMEM((CHUNK,), jnp.float32),
            pltpu.VMEM((CHUNK,), jnp.int32),
            pltpu.VMEM((LANES,), jnp.float32),
            pltpu.VMEM((LANES,), jnp.float32),
            pltpu.SemaphoreType.DMA,
        ],
    )
    def body(logits_hbm, counts_hbm, conf_hbm, maxl_hbm, out_hbm,
             lv, cv, pv, mv, sem):
        base = lax.axis_index("s") * CHUNK
        c1 = pltpu.async_copy(logits_hbm.at[pl.ds(base, CHUNK)], lv, sem)
        c2 = pltpu.async_copy(counts_hbm.at[pl.ds(base, CHUNK)], cv, sem)
        c3 = pltpu.async_copy(conf_hbm, pv.at[pl.ds(0, MAX_IDX + 1)], sem)
        c4 = pltpu.async_copy(maxl_hbm, mv.at[pl.ds(0, 1)], sem)
        c1.wait()
        c2.wait()
        c3.wait()
        c4.wait()
        params = pv[...]
        lane = lax.iota(jnp.int32, LANES)
        m = _vgather(mv[...], (lane * 0)[:, None])
        two_inv_m = 2.0 / m
        two_m = m + m

        def step(i, carry):
            sl = pl.ds(i * LANES, LANES)
            idx = jnp.minimum(cv[sl], MAX_IDX)
            scale = _vgather(params, idx[:, None])
            e = jnp.exp(lv[sl] * scale * two_inv_m)
            lv[sl] = m - two_m / (e + 1.0)
            return carry

        lax.fori_loop(0, NUM_VECS, step, 0)
        pltpu.sync_copy(lv, out_hbm.at[pl.ds(base, CHUNK)])

    return body


_calibrate = _make_sc_kernel()


def kernel(logits, alt_counts, confidence, max_logit):
    counts = alt_counts.astype(jnp.int32)
    maxl = jnp.reshape(max_logit, (1,)).astype(jnp.float32)
    return _calibrate(logits.astype(jnp.float32), counts,
                      confidence.astype(jnp.float32), maxl)


# speedup vs baseline: 1.0789x; 1.0774x over previous
import functools

import jax
import jax.numpy as jnp
from jax import lax
from jax.experimental import pallas as pl
from jax.experimental.pallas import tpu as pltpu
from jax.experimental.pallas import tpu_sc as plsc

MAX_IDX = 10
N = 16384
LANES = 16
NUM_SUBCORES = 16
CHUNK = N // NUM_SUBCORES
NUM_VECS = CHUNK // LANES

_GATHER_DNUMS = lax.GatherDimensionNumbers(
    offset_dims=(), collapsed_slice_dims=(0,), start_index_map=(0,))


def _vgather(vec, idx):
    return lax.gather(vec, idx, _GATHER_DNUMS, slice_sizes=(1,),
                      mode=lax.GatherScatterMode.PROMISE_IN_BOUNDS)


def _make_sc_kernel():
    mesh = plsc.VectorSubcoreMesh(core_axis_name="c", subcore_axis_name="s",
                                  num_cores=1)

    @functools.partial(
        pl.kernel,
        mesh=mesh,
        out_type=jax.ShapeDtypeStruct((N,), jnp.float32),
        scratch_types=[
            pltpu.VMEM((CHUNK,), jnp.float32),
            pltpu.VMEM((CHUNK,), jnp.int32),
            pltpu.VMEM((CHUNK,), jnp.float32),
            pltpu.VMEM((LANES,), jnp.float32),
            pltpu.VMEM((LANES,), jnp.float32),
            pltpu.SemaphoreType.DMA,
        ],
    )
    def body(logits_hbm, counts_hbm, conf_hbm, maxl_hbm, out_hbm,
             lv, cv, ov, pv, mv, sem):
        base = lax.axis_index("s") * CHUNK
        c1 = pltpu.async_copy(logits_hbm.at[pl.ds(base, CHUNK)], lv, sem)
        c2 = pltpu.async_copy(counts_hbm.at[pl.ds(base, CHUNK)], cv, sem)
        c3 = pltpu.async_copy(conf_hbm, pv.at[pl.ds(0, MAX_IDX + 1)], sem)
        c4 = pltpu.async_copy(maxl_hbm, mv.at[pl.ds(0, 1)], sem)
        c1.wait()
        c2.wait()
        c3.wait()
        c4.wait()
        params = pv[...]
        lane = lax.iota(jnp.int32, LANES)
        m = _vgather(mv[...], (lane * 0)[:, None])
        two_inv_m = 2.0 / m
        two_m = m + m

        def step(i, carry):
            sl = pl.ds(i * LANES, LANES)
            idx = jnp.minimum(cv[sl], MAX_IDX)
            scale = _vgather(params, idx[:, None])
            e = jnp.exp(lv[sl] * scale * two_inv_m)
            ov[sl] = m - two_m / (e + 1.0)
            return carry

        lax.fori_loop(0, NUM_VECS, step, 0)
        pltpu.sync_copy(ov, out_hbm.at[pl.ds(base, CHUNK)])

    return body


_calibrate = _make_sc_kernel()


def kernel(logits, alt_counts, confidence, max_logit):
    counts = alt_counts.astype(jnp.int32)
    maxl = jnp.reshape(max_logit, (1,)).astype(jnp.float32)
    return _calibrate(logits.astype(jnp.float32), counts,
                      confidence.astype(jnp.float32), maxl)
